# Initial kernel scaffold; baseline (speedup 1.0000x reference)
import jax
import jax.numpy as jnp
from jax.experimental import pallas as pl


def kernel(input, encoder_weight):
    raise NotImplementedError("write your pallas kernel here")


# speedup vs baseline: 1.8584x; 1.8584x over previous
import functools

import jax
import jax.numpy as jnp
from jax import lax
from jax.experimental import pallas as pl
from jax.experimental.pallas import tpu as pltpu
from jax.experimental.pallas import tpu_sc as plsc

_NTOKEN = 1000000
_NINP = 64
_BATCH = 16384
_HIST = 50
_B_TOTAL = _BATCH * _HIST
_NW = 32
_B_PER_W = _B_TOTAL // _NW
_CHUNK = 1024
_N_CHUNKS = _B_PER_W // _CHUNK


def _emb_body(idx_hbm, table_hbm, out_hbm, idx_v, rows_v, sem):
    wid = lax.axis_index("s") * 2 + lax.axis_index("c")
    base = wid * _B_PER_W

    def body(g, carry):
        off = base + g * _CHUNK
        pltpu.sync_copy(idx_hbm.at[pl.ds(off, _CHUNK)], idx_v)
        pltpu.async_copy(table_hbm.at[idx_v], rows_v, sem).wait()
        pltpu.sync_copy(rows_v, out_hbm.at[pl.ds(off, _CHUNK)])
        return carry

    lax.fori_loop(0, _N_CHUNKS, body, 0)


_mesh = plsc.VectorSubcoreMesh(core_axis_name="c", subcore_axis_name="s")


@jax.jit
def _run(idx_flat, table):
    return pl.kernel(
        _emb_body,
        out_type=jax.ShapeDtypeStruct((_B_TOTAL, _NINP), jnp.float32),
        mesh=_mesh,
        scratch_types=[
            pltpu.VMEM((_CHUNK,), jnp.int32),
            pltpu.VMEM((_CHUNK, _NINP), jnp.float32),
            pltpu.SemaphoreType.DMA,
        ],
        compiler_params=pltpu.CompilerParams(use_tc_tiling_on_sc=False),
    )(idx_flat, table)


def kernel(input, encoder_weight):
    idx_flat = input.reshape(-1).astype(jnp.int32)
    out = _run(idx_flat, encoder_weight)
    return out.reshape(_BATCH, _HIST, _NINP)


# speedup vs baseline: 1.8696x; 1.0060x over previous
import functools

import jax
import jax.numpy as jnp
from jax import lax
from jax.experimental import pallas as pl
from jax.experimental.pallas import tpu as pltpu
from jax.experimental.pallas import tpu_sc as plsc

_NTOKEN = 1000000
_NINP = 64
_BATCH = 16384
_HIST = 50
_B_TOTAL = _BATCH * _HIST
_NW = 32
_B_PER_W = _B_TOTAL // _NW
_CHUNK = 800
_N_CHUNKS = _B_PER_W // _CHUNK


def _emb_body(idx_hbm, table_hbm, out_hbm,
              idx0, idx1, rows0, rows1, si0, si1, sg, sw0, sw1):
    idx_v = (idx0, idx1)
    rows_v = (rows0, rows1)
    si = (si0, si1)
    sw = (sw0, sw1)

    wid = lax.axis_index("s") * 2 + lax.axis_index("c")
    base = wid * _B_PER_W

    def start_idx(g, b):
        pltpu.async_copy(idx_hbm.at[pl.ds(base + g * _CHUNK, _CHUNK)],
                         idx_v[b], si[b])

    def wait_idx(b):
        pltpu.make_async_copy(idx_hbm.at[pl.ds(0, _CHUNK)], idx_v[b],
                              si[b]).wait()

    def start_write(g, b):
        pltpu.async_copy(rows_v[b],
                         out_hbm.at[pl.ds(base + g * _CHUNK, _CHUNK)], sw[b])

    def wait_write(b):
        pltpu.make_async_copy(rows_v[b], out_hbm.at[pl.ds(0, _CHUNK)],
                              sw[b]).wait()

    def gather(b):
        pltpu.async_copy(table_hbm.at[idx_v[b]], rows_v[b], sg).wait()

    start_idx(0, 0)
    start_idx(1, 1)
    for b in range(2):
        wait_idx(b)
        gather(b)
        start_idx(b + 2, b)
        start_write(b, b)

    def pair_body(i, carry):
        for b in range(2):
            g = 2 * i + b
            wait_idx(b)
            wait_write(b)
            gather(b)
            gp = jnp.minimum(g + 2, _N_CHUNKS - 1)
            start_idx(gp, b)
            start_write(g, b)
        return carry

    lax.fori_loop(1, _N_CHUNKS // 2, pair_body, 0)

    for b in range(2):
        wait_idx(b)
        wait_write(b)


_mesh = plsc.VectorSubcoreMesh(core_axis_name="c", subcore_axis_name="s")


@jax.jit
def _run(idx_flat, table):
    return pl.kernel(
        _emb_body,
        out_type=jax.ShapeDtypeStruct((_B_TOTAL, _NINP), jnp.float32),
        mesh=_mesh,
        scratch_types=[
            pltpu.VMEM((_CHUNK,), jnp.int32),
            pltpu.VMEM((_CHUNK,), jnp.int32),
            pltpu.VMEM((_CHUNK, _NINP), jnp.float32),
            pltpu.VMEM((_CHUNK, _NINP), jnp.float32),
            pltpu.SemaphoreType.DMA,
            pltpu.SemaphoreType.DMA,
            pltpu.SemaphoreType.DMA,
            pltpu.SemaphoreType.DMA,
            pltpu.SemaphoreType.DMA,
        ],
        compiler_params=pltpu.CompilerParams(use_tc_tiling_on_sc=False),
    )(idx_flat, table)


def kernel(input, encoder_weight):
    idx_flat = input.reshape(-1).astype(jnp.int32)
    out = _run(idx_flat, encoder_weight)
    return out.reshape(_BATCH, _HIST, _NINP)


# speedup vs baseline: 1.8710x; 1.0008x over previous
import functools

import jax
import jax.numpy as jnp
from jax import lax
from jax.experimental import pallas as pl
from jax.experimental.pallas import tpu as pltpu
from jax.experimental.pallas import tpu_sc as plsc

_NTOKEN = 1000000
_NINP = 64
_BATCH = 16384
_HIST = 50
_B_TOTAL = _BATCH * _HIST
_NW = 32
_B_PER_W = _B_TOTAL // _NW
_CHUNK = 800
_N_CHUNKS = _B_PER_W // _CHUNK
_CB = _CHUNK // _HIST


def _emb_body(idx_hbm, table_hbm, out3d_hbm,
              idx0, idx1, rows0, rows1, si0, si1, sg, sw0, sw1):
    out_hbm = out3d_hbm
    idx_v = (idx0, idx1)
    rows_v = (rows0, rows1)
    si = (si0, si1)
    sw = (sw0, sw1)

    wid = lax.axis_index("s") * 2 + lax.axis_index("c")
    base = wid * _B_PER_W

    def start_idx(g, b):
        pltpu.async_copy(idx_hbm.at[pl.ds(base + g * _CHUNK, _CHUNK)],
                         idx_v[b], si[b])

    def wait_idx(b):
        pltpu.make_async_copy(idx_hbm.at[pl.ds(0, _CHUNK)], idx_v[b],
                              si[b]).wait()

    def start_write(g, b):
        r0 = (base + g * _CHUNK) // _HIST
        for k in range(_CB):
            pltpu.async_copy(rows_v[b].at[pl.ds(k * _HIST, _HIST), :],
                             out_hbm.at[r0 + k], sw[b])

    def wait_write(b):
        for k in range(_CB):
            pltpu.make_async_copy(rows_v[b].at[pl.ds(0, _HIST), :],
                                  out_hbm.at[0], sw[b]).wait()

    def gather(b):
        pltpu.async_copy(table_hbm.at[idx_v[b]], rows_v[b], sg).wait()

    start_idx(0, 0)
    start_idx(1, 1)
    for b in range(2):
        wait_idx(b)
        gather(b)
        start_idx(b + 2, b)
        start_write(b, b)

    def pair_body(i, carry):
        for b in range(2):
            g = 2 * i + b
            wait_idx(b)
            wait_write(b)
            gather(b)
            gp = jnp.minimum(g + 2, _N_CHUNKS - 1)
            start_idx(gp, b)
            start_write(g, b)
        return carry

    lax.fori_loop(1, _N_CHUNKS // 2, pair_body, 0)

    for b in range(2):
        wait_idx(b)
        wait_write(b)


_mesh = plsc.VectorSubcoreMesh(core_axis_name="c", subcore_axis_name="s")


@jax.jit
def _run(idx_flat, table):
    return pl.kernel(
        _emb_body,
        out_type=jax.ShapeDtypeStruct((_BATCH, _HIST, _NINP), jnp.float32),
        mesh=_mesh,
        scratch_types=[
            pltpu.VMEM((_CHUNK,), jnp.int32),
            pltpu.VMEM((_CHUNK,), jnp.int32),
            pltpu.VMEM((_CHUNK, _NINP), jnp.float32),
            pltpu.VMEM((_CHUNK, _NINP), jnp.float32),
            pltpu.SemaphoreType.DMA,
            pltpu.SemaphoreType.DMA,
            pltpu.SemaphoreType.DMA,
            pltpu.SemaphoreType.DMA,
            pltpu.SemaphoreType.DMA,
        ],
        compiler_params=pltpu.CompilerParams(use_tc_tiling_on_sc=False),
    )(idx_flat, table)


def kernel(input, encoder_weight):
    idx_flat = input.reshape(-1)
    return _run(idx_flat, encoder_weight)
